# initial kernel scaffold (unmeasured)
import numpy as np
import jax
import jax.numpy as jnp
from jax import lax
from jax.experimental import pallas as pl
from jax.experimental.pallas import tpu as pltpu

N_DEV = 16
B, SQ, DMODEL = 2, 256, 512
HQ_PER, DH = 4, 64
SKV_PER = 256
SQ_PER = SQ // N_DEV


def _chunk_mask(s: int) -> np.ndarray:
    qb = np.arange(SQ) // 64
    kb = 4 * s + np.arange(SKV_PER) // 64
    return (
        (qb[:, None] == kb[None, :])
        | (kb[None, :] == 0)
        | ((qb[:, None] + kb[None, :]) % 3 == 0)
    )


def kernel(x, Wq, K_ext, V_ext, Wo):
    def body(x_ref, wq_ref, k_ref, v_ref, wo_ref, out_ref,
             kv_send, kv_recv, rs_send, rs_recv, ag_send, ag_recv,
             kv_ssem, kv_rsem, rs_ssem, rs_rsem, ag_ssem, ag_rsem,
             loc_sem):
        me = lax.axis_index("i")

        bar = pltpu.get_barrier_semaphore()
        for k in range(1, N_DEV):
            pl.semaphore_signal(
                bar, inc=1, device_id=((me + k) % N_DEV,),
                device_id_type=pl.DeviceIdType.MESH,
            )
        pl.semaphore_wait(bar, N_DEV - 1)

        Kt = k_ref[...].reshape(B, SKV_PER, N_DEV, HQ_PER, DH).transpose(2, 0, 1, 3, 4)
        Vt = v_ref[...].reshape(B, SKV_PER, N_DEV, HQ_PER, DH).transpose(2, 0, 1, 3, 4)
        kv_send[...] = jnp.stack([Kt, Vt], axis=1)

        cp = pltpu.make_async_copy(kv_send.at[me], kv_recv.at[me], loc_sem)
        cp.start()
        cp.wait()

        kv_rdmas = []
        for k in range(1, N_DEV):
            d = (me + k) % N_DEV
            r = pltpu.make_async_remote_copy(
                src_ref=kv_send.at[d], dst_ref=kv_recv.at[me],
                send_sem=kv_ssem.at[k - 1], recv_sem=kv_rsem.at[k - 1],
                device_id=(d,), device_id_type=pl.DeviceIdType.MESH,
            )
            r.start()
            kv_rdmas.append(r)
        for r in kv_rdmas:
            r.wait_send()
        for k in range(1, N_DEV):
            s = (me - k) % N_DEV
            pltpu.make_async_remote_copy(
                src_ref=kv_send.at[s], dst_ref=kv_recv.at[s],
                send_sem=kv_ssem.at[k - 1], recv_sem=kv_rsem.at[k - 1],
                device_id=(me,), device_id_type=pl.DeviceIdType.MESH,
            ).wait_recv()

        xq = x_ref[...].reshape(B * SQ, DMODEL) @ wq_ref[...]
        Qh = xq.reshape(B, SQ, HQ_PER, DH).transpose(0, 2, 1, 3)

        m_run = jnp.full((B, HQ_PER, SQ, 1), -1e30, jnp.float32)
        l_run = jnp.zeros((B, HQ_PER, SQ, 1), jnp.float32)
        acc = jnp.zeros((B, HQ_PER, SQ, DH), jnp.float32)
        for s in range(N_DEV):
            Kc = kv_recv[s, 0].transpose(0, 2, 1, 3)
            Vc = kv_recv[s, 1].transpose(0, 2, 1, 3)
            sc = lax.dot_general(
                Qh, Kc, (((3,), (3,)), ((0, 1), (0, 1)))
            ) * 0.125
            msk = jnp.asarray(_chunk_mask(s))
            sc = jnp.where(msk[None, None], sc, -1e9)
            m_new = jnp.maximum(m_run, sc.max(axis=-1, keepdims=True))
            alpha = jnp.exp(m_run - m_new)
            p = jnp.exp(sc - m_new)
            l_run = l_run * alpha + p.sum(axis=-1, keepdims=True)
            acc = acc * alpha + lax.dot_general(
                p, Vc, (((3,), (2,)), ((0, 1), (0, 1)))
            )
            m_run = m_new

        ctx = (acc / l_run).transpose(0, 2, 1, 3).reshape(B * SQ, HQ_PER * DH)
        P = ctx @ wo_ref[...]

        rs_send[...] = P.reshape(B, N_DEV, SQ_PER, DMODEL).transpose(1, 0, 2, 3)
        cp = pltpu.make_async_copy(rs_send.at[me], rs_recv.at[me], loc_sem)
        cp.start()
        cp.wait()
        rs_rdmas = []
        for k in range(1, N_DEV):
            d = (me + k) % N_DEV
            r = pltpu.make_async_remote_copy(
                src_ref=rs_send.at[d], dst_ref=rs_recv.at[me],
                send_sem=rs_ssem.at[k - 1], recv_sem=rs_rsem.at[k - 1],
                device_id=(d,), device_id_type=pl.DeviceIdType.MESH,
            )
            r.start()
            rs_rdmas.append(r)
        for r in rs_rdmas:
            r.wait_send()
        for k in range(1, N_DEV):
            s = (me - k) % N_DEV
            pltpu.make_async_remote_copy(
                src_ref=rs_send.at[s], dst_ref=rs_recv.at[s],
                send_sem=rs_ssem.at[k - 1], recv_sem=rs_rsem.at[k - 1],
                device_id=(me,), device_id_type=pl.DeviceIdType.MESH,
            ).wait_recv()
        Rsum = rs_recv[...].sum(axis=0)

        ag_send[...] = Rsum
        cp = pltpu.make_async_copy(ag_send, ag_recv.at[me], loc_sem)
        cp.start()
        cp.wait()
        ag_rdmas = []
        for k in range(1, N_DEV):
            d = (me + k) % N_DEV
            r = pltpu.make_async_remote_copy(
                src_ref=ag_send, dst_ref=ag_recv.at[me],
                send_sem=ag_ssem.at[k - 1], recv_sem=ag_rsem.at[k - 1],
                device_id=(d,), device_id_type=pl.DeviceIdType.MESH,
            )
            r.start()
            ag_rdmas.append(r)
        for r in ag_rdmas:
            r.wait_send()
        for k in range(1, N_DEV):
            s = (me - k) % N_DEV
            pltpu.make_async_remote_copy(
                src_ref=ag_send, dst_ref=ag_recv.at[s],
                send_sem=ag_ssem.at[k - 1], recv_sem=ag_rsem.at[k - 1],
                device_id=(me,), device_id_type=pl.DeviceIdType.MESH,
            ).wait_recv()

        out_ref[...] = ag_recv[...].transpose(1, 0, 2, 3).reshape(B, SQ, DMODEL)

    out_shape = jax.ShapeDtypeStruct((B, SQ, DMODEL), jnp.float32)
    return pl.pallas_call(
        body,
        out_shape=out_shape,
        in_specs=[pl.BlockSpec(memory_space=pltpu.VMEM)] * 5,
        out_specs=pl.BlockSpec(memory_space=pltpu.VMEM),
        scratch_shapes=[
            pltpu.VMEM((N_DEV, 2, B, SKV_PER, HQ_PER, DH), jnp.float32),
            pltpu.VMEM((N_DEV, 2, B, SKV_PER, HQ_PER, DH), jnp.float32),
            pltpu.VMEM((N_DEV, B, SQ_PER, DMODEL), jnp.float32),
            pltpu.VMEM((N_DEV, B, SQ_PER, DMODEL), jnp.float32),
            pltpu.VMEM((B, SQ_PER, DMODEL), jnp.float32),
            pltpu.VMEM((N_DEV, B, SQ_PER, DMODEL), jnp.float32),
            pltpu.SemaphoreType.DMA((N_DEV - 1,)),
            pltpu.SemaphoreType.DMA((N_DEV - 1,)),
            pltpu.SemaphoreType.DMA((N_DEV - 1,)),
            pltpu.SemaphoreType.DMA((N_DEV - 1,)),
            pltpu.SemaphoreType.DMA((N_DEV - 1,)),
            pltpu.SemaphoreType.DMA((N_DEV - 1,)),
            pltpu.SemaphoreType.DMA,
        ],
        compiler_params=pltpu.CompilerParams(collective_id=0),
    )(x, Wq, K_ext, V_ext, Wo)


# baseline (device time: 340591 ns/iter reference)
import numpy as np
import jax
import jax.numpy as jnp
from jax import lax
from jax.experimental import pallas as pl
from jax.experimental.pallas import tpu as pltpu

N_DEV = 16
B, SQ, DMODEL = 2, 256, 512
HQ_PER, DH = 4, 64
SKV_PER = 256
SQ_PER = SQ // N_DEV


def _chunk_mask(s: int) -> np.ndarray:
    qb = np.arange(SQ) // 64
    kb = 4 * s + np.arange(SKV_PER) // 64
    return (
        (qb[:, None] == kb[None, :])
        | (kb[None, :] == 0)
        | ((qb[:, None] + kb[None, :]) % 3 == 0)
    )


def kernel(x, Wq, K_ext, V_ext, Wo):
    def body(x_ref, wq_ref, k_ref, v_ref, wo_ref, out_ref,
             kv_send, kv_recv, rs_send, rs_recv, ag_send, ag_recv,
             kv_ssem, kv_rsem, rs_ssem, rs_rsem, ag_ssem, ag_rsem,
             loc_sem):
        me = lax.axis_index("i")

        bar = pltpu.get_barrier_semaphore()
        for k in range(1, N_DEV):
            pl.semaphore_signal(
                bar, inc=1, device_id=((me + k) % N_DEV,),
                device_id_type=pl.DeviceIdType.MESH,
            )
        pl.semaphore_wait(bar, N_DEV - 1)

        for d in range(N_DEV):
            cp = pltpu.make_async_copy(
                k_ref.at[:, :, pl.ds(256 * d, 256)], kv_send.at[d, 0], loc_sem)
            cp.start()
            cp.wait()
            cp = pltpu.make_async_copy(
                v_ref.at[:, :, pl.ds(256 * d, 256)], kv_send.at[d, 1], loc_sem)
            cp.start()
            cp.wait()

        cp = pltpu.make_async_copy(kv_send.at[me], kv_recv.at[me], loc_sem)
        cp.start()
        cp.wait()

        kv_rdmas = []
        for k in range(1, N_DEV):
            d = (me + k) % N_DEV
            r = pltpu.make_async_remote_copy(
                src_ref=kv_send.at[d], dst_ref=kv_recv.at[me],
                send_sem=kv_ssem.at[k - 1], recv_sem=kv_rsem.at[k - 1],
                device_id=(d,), device_id_type=pl.DeviceIdType.MESH,
            )
            r.start()
            kv_rdmas.append(r)
        for r in kv_rdmas:
            r.wait_send()
        for k in range(1, N_DEV):
            s = (me - k) % N_DEV
            pltpu.make_async_remote_copy(
                src_ref=kv_send.at[s], dst_ref=kv_recv.at[s],
                send_sem=kv_ssem.at[k - 1], recv_sem=kv_rsem.at[k - 1],
                device_id=(me,), device_id_type=pl.DeviceIdType.MESH,
            ).wait_recv()

        xq = x_ref[...].reshape(B * SQ, DMODEL) @ wq_ref[...]
        BH = B * HQ_PER
        Qh = xq.reshape(B, SQ, HQ_PER, DH).transpose(0, 2, 1, 3).reshape(BH, SQ, DH)

        m_run = jnp.full((BH, SQ, 1), -1e30, jnp.float32)
        l_run = jnp.zeros((BH, SQ, 1), jnp.float32)
        acc = jnp.zeros((BH, SQ, DH), jnp.float32)
        for s in range(N_DEV):
            Kc = (kv_recv[s, 0].reshape(B, SKV_PER, HQ_PER, DH)
                  .transpose(0, 2, 1, 3).reshape(BH, SKV_PER, DH))
            Vc = (kv_recv[s, 1].reshape(B, SKV_PER, HQ_PER, DH)
                  .transpose(0, 2, 1, 3).reshape(BH, SKV_PER, DH))
            sc = lax.dot_general(
                Qh, Kc, (((2,), (2,)), ((0,), (0,)))
            ) * 0.125
            qb = lax.broadcasted_iota(jnp.int32, (SQ, SKV_PER), 0) // 64
            kb = 4 * s + lax.broadcasted_iota(jnp.int32, (SQ, SKV_PER), 1) // 64
            msk = (qb == kb) | (kb == 0) | ((qb + kb) % 3 == 0)
            sc = jnp.where(msk[None], sc, -1e9)
            m_new = jnp.maximum(m_run, sc.max(axis=-1, keepdims=True))
            alpha = jnp.exp(m_run - m_new)
            p = jnp.exp(sc - m_new)
            l_run = l_run * alpha + p.sum(axis=-1, keepdims=True)
            acc = acc * alpha + lax.dot_general(
                p, Vc, (((2,), (1,)), ((0,), (0,)))
            )
            m_run = m_new

        ctx = (acc / l_run).reshape(B, HQ_PER, SQ, DH)
        ctx = ctx.transpose(0, 2, 1, 3).reshape(B * SQ, HQ_PER * DH)
        P = ctx @ wo_ref[...]

        rs_send[...] = P.reshape(B, N_DEV, SQ_PER, DMODEL).transpose(1, 0, 2, 3)
        cp = pltpu.make_async_copy(rs_send.at[me], rs_recv.at[me], loc_sem)
        cp.start()
        cp.wait()
        rs_rdmas = []
        for k in range(1, N_DEV):
            d = (me + k) % N_DEV
            r = pltpu.make_async_remote_copy(
                src_ref=rs_send.at[d], dst_ref=rs_recv.at[me],
                send_sem=rs_ssem.at[k - 1], recv_sem=rs_rsem.at[k - 1],
                device_id=(d,), device_id_type=pl.DeviceIdType.MESH,
            )
            r.start()
            rs_rdmas.append(r)
        for r in rs_rdmas:
            r.wait_send()
        for k in range(1, N_DEV):
            s = (me - k) % N_DEV
            pltpu.make_async_remote_copy(
                src_ref=rs_send.at[s], dst_ref=rs_recv.at[s],
                send_sem=rs_ssem.at[k - 1], recv_sem=rs_rsem.at[k - 1],
                device_id=(me,), device_id_type=pl.DeviceIdType.MESH,
            ).wait_recv()
        Rsum = rs_recv[...].sum(axis=0)

        ag_send[...] = Rsum
        cp = pltpu.make_async_copy(ag_send, ag_recv.at[me], loc_sem)
        cp.start()
        cp.wait()
        ag_rdmas = []
        for k in range(1, N_DEV):
            d = (me + k) % N_DEV
            r = pltpu.make_async_remote_copy(
                src_ref=ag_send, dst_ref=ag_recv.at[me],
                send_sem=ag_ssem.at[k - 1], recv_sem=ag_rsem.at[k - 1],
                device_id=(d,), device_id_type=pl.DeviceIdType.MESH,
            )
            r.start()
            ag_rdmas.append(r)
        for r in ag_rdmas:
            r.wait_send()
        for k in range(1, N_DEV):
            s = (me - k) % N_DEV
            pltpu.make_async_remote_copy(
                src_ref=ag_send, dst_ref=ag_recv.at[s],
                send_sem=ag_ssem.at[k - 1], recv_sem=ag_rsem.at[k - 1],
                device_id=(me,), device_id_type=pl.DeviceIdType.MESH,
            ).wait_recv()

        out_ref[...] = ag_recv[...].transpose(1, 0, 2, 3).reshape(B, SQ, DMODEL)

    out_shape = jax.ShapeDtypeStruct((B, SQ, DMODEL), jnp.float32)
    return pl.pallas_call(
        body,
        out_shape=out_shape,
        in_specs=[
            pl.BlockSpec(memory_space=pltpu.VMEM),
            pl.BlockSpec(memory_space=pltpu.VMEM),
            pl.BlockSpec(memory_space=pl.ANY),
            pl.BlockSpec(memory_space=pl.ANY),
            pl.BlockSpec(memory_space=pltpu.VMEM),
        ],
        out_specs=pl.BlockSpec(memory_space=pltpu.VMEM),
        scratch_shapes=[
            pltpu.VMEM((N_DEV, 2, B, SKV_PER, HQ_PER * DH), jnp.float32),
            pltpu.VMEM((N_DEV, 2, B, SKV_PER, HQ_PER * DH), jnp.float32),
            pltpu.VMEM((N_DEV, B, SQ_PER, DMODEL), jnp.float32),
            pltpu.VMEM((N_DEV, B, SQ_PER, DMODEL), jnp.float32),
            pltpu.VMEM((B, SQ_PER, DMODEL), jnp.float32),
            pltpu.VMEM((N_DEV, B, SQ_PER, DMODEL), jnp.float32),
            pltpu.SemaphoreType.DMA((N_DEV - 1,)),
            pltpu.SemaphoreType.DMA((N_DEV - 1,)),
            pltpu.SemaphoreType.DMA((N_DEV - 1,)),
            pltpu.SemaphoreType.DMA((N_DEV - 1,)),
            pltpu.SemaphoreType.DMA((N_DEV - 1,)),
            pltpu.SemaphoreType.DMA((N_DEV - 1,)),
            pltpu.SemaphoreType.DMA,
        ],
        compiler_params=pltpu.CompilerParams(
            collective_id=0, vmem_limit_bytes=63 * 1024 * 1024
        ),
    )(
        x, Wq,
        K_ext.reshape(B, SKV_PER, 64 * DH),
        V_ext.reshape(B, SKV_PER, 64 * DH),
        Wo,
    )


# device time: 316880 ns/iter; 1.0748x vs baseline; 1.0748x over previous
import numpy as np
import jax
import jax.numpy as jnp
from jax import lax
from jax.experimental import pallas as pl
from jax.experimental.pallas import tpu as pltpu

N_DEV = 16
B, SQ, DMODEL = 2, 256, 512
HQ_PER, DH = 4, 64
SKV_PER = 256
SQ_PER = SQ // N_DEV


def _chunk_mask(s: int) -> np.ndarray:
    qb = np.arange(SQ) // 64
    kb = 4 * s + np.arange(SKV_PER) // 64
    return (
        (qb[:, None] == kb[None, :])
        | (kb[None, :] == 0)
        | ((qb[:, None] + kb[None, :]) % 3 == 0)
    )


def kernel(x, Wq, K_ext, V_ext, Wo):
    def body(x_ref, wq_ref, k_ref, v_ref, wo_ref, out_ref,
             kv_send, kv_recv, rs_send, rs_recv, ag_send, ag_recv,
             kv_ssem, kv_rsem, rs_ssem, rs_rsem, ag_ssem, ag_rsem,
             pack_sems, loc_sem):
        me = lax.axis_index("i")

        bar = pltpu.get_barrier_semaphore()
        for k in range(1, N_DEV):
            pl.semaphore_signal(
                bar, inc=1, device_id=((me + k) % N_DEV,),
                device_id_type=pl.DeviceIdType.MESH,
            )
        pl.semaphore_wait(bar, N_DEV - 1)

        packs = []
        for k in range(N_DEV):
            d = (me + k) % N_DEV
            cpk = pltpu.make_async_copy(
                k_ref.at[:, :, pl.ds(256 * d, 256)], kv_send.at[d, 0],
                pack_sems.at[k])
            cpv = pltpu.make_async_copy(
                v_ref.at[:, :, pl.ds(256 * d, 256)], kv_send.at[d, 1],
                pack_sems.at[k])
            cpk.start()
            cpv.start()
            packs.append((cpk, cpv))

        xq = x_ref[...].reshape(B * SQ, DMODEL) @ wq_ref[...]
        BH = B * HQ_PER
        Qh = xq.reshape(B, SQ, HQ_PER, DH).transpose(0, 2, 1, 3).reshape(BH, SQ, DH)

        kv_rdmas = []
        for k in range(1, N_DEV):
            d = (me + k) % N_DEV
            packs[k][0].wait()
            packs[k][1].wait()
            r = pltpu.make_async_remote_copy(
                src_ref=kv_send.at[d], dst_ref=kv_recv.at[me],
                send_sem=kv_ssem.at[k - 1], recv_sem=kv_rsem.at[k - 1],
                device_id=(d,), device_id_type=pl.DeviceIdType.MESH,
            )
            r.start()
            kv_rdmas.append(r)

        packs[0][0].wait()
        packs[0][1].wait()
        cp = pltpu.make_async_copy(kv_send.at[me], kv_recv.at[me], loc_sem)
        cp.start()
        cp.wait()

        m_run = jnp.full((BH, SQ, 1), -1e30, jnp.float32)
        l_run = jnp.zeros((BH, SQ, 1), jnp.float32)
        acc = jnp.zeros((BH, SQ, DH), jnp.float32)

        def flash_chunk(s, carry):
            m_run, l_run, acc = carry
            Kc = (kv_recv[s, 0]
                  .reshape(B, SKV_PER, HQ_PER, DH)
                  .transpose(0, 2, 1, 3).reshape(BH, SKV_PER, DH))
            Vc = (kv_recv[s, 1]
                  .reshape(B, SKV_PER, HQ_PER, DH)
                  .transpose(0, 2, 1, 3).reshape(BH, SKV_PER, DH))
            sc = lax.dot_general(
                Qh, Kc, (((2,), (2,)), ((0,), (0,)))
            ) * 0.125
            qb = lax.broadcasted_iota(jnp.int32, (SQ, SKV_PER), 0) // 64
            kb = 4 * s + lax.broadcasted_iota(jnp.int32, (SQ, SKV_PER), 1) // 64
            msk = (qb == kb) | (kb == 0) | ((qb + kb) % 3 == 0)
            sc = jnp.where(msk[None], sc, -1e9)
            m_new = jnp.maximum(m_run, sc.max(axis=-1, keepdims=True))
            alpha = jnp.exp(m_run - m_new)
            p = jnp.exp(sc - m_new)
            l_new = l_run * alpha + p.sum(axis=-1, keepdims=True)
            acc_new = acc * alpha + lax.dot_general(
                p, Vc, (((2,), (1,)), ((0,), (0,)))
            )
            return m_new, l_new, acc_new

        for k in range(1, N_DEV):
            s = (me - k) % N_DEV
            pltpu.make_async_remote_copy(
                src_ref=kv_send.at[s], dst_ref=kv_recv.at[s],
                send_sem=kv_ssem.at[k - 1], recv_sem=kv_rsem.at[k - 1],
                device_id=(me,), device_id_type=pl.DeviceIdType.MESH,
            ).wait_recv()
        carry = (m_run, l_run, acc)
        for s in range(N_DEV):
            carry = flash_chunk(s, carry)
        m_run, l_run, acc = carry
        for r in kv_rdmas:
            r.wait_send()

        ctx = (acc / l_run).reshape(B, HQ_PER, SQ, DH)
        ctx = ctx.transpose(0, 2, 1, 3).reshape(B * SQ, HQ_PER * DH)
        P = ctx @ wo_ref[...]

        rs_send[...] = P.reshape(B, N_DEV, SQ_PER, DMODEL).transpose(1, 0, 2, 3)
        cp = pltpu.make_async_copy(rs_send.at[me], rs_recv.at[me], loc_sem)
        cp.start()
        cp.wait()
        rs_rdmas = []
        for k in range(1, N_DEV):
            d = (me + k) % N_DEV
            r = pltpu.make_async_remote_copy(
                src_ref=rs_send.at[d], dst_ref=rs_recv.at[me],
                send_sem=rs_ssem.at[k - 1], recv_sem=rs_rsem.at[k - 1],
                device_id=(d,), device_id_type=pl.DeviceIdType.MESH,
            )
            r.start()
            rs_rdmas.append(r)
        for r in rs_rdmas:
            r.wait_send()
        for k in range(1, N_DEV):
            s = (me - k) % N_DEV
            pltpu.make_async_remote_copy(
                src_ref=rs_send.at[s], dst_ref=rs_recv.at[s],
                send_sem=rs_ssem.at[k - 1], recv_sem=rs_rsem.at[k - 1],
                device_id=(me,), device_id_type=pl.DeviceIdType.MESH,
            ).wait_recv()
        Rsum = rs_recv[...].sum(axis=0)

        ag_send[...] = Rsum
        cp = pltpu.make_async_copy(ag_send, ag_recv.at[me], loc_sem)
        cp.start()
        cp.wait()
        ag_rdmas = []
        for k in range(1, N_DEV):
            d = (me + k) % N_DEV
            r = pltpu.make_async_remote_copy(
                src_ref=ag_send, dst_ref=ag_recv.at[me],
                send_sem=ag_ssem.at[k - 1], recv_sem=ag_rsem.at[k - 1],
                device_id=(d,), device_id_type=pl.DeviceIdType.MESH,
            )
            r.start()
            ag_rdmas.append(r)
        for r in ag_rdmas:
            r.wait_send()
        for k in range(1, N_DEV):
            s = (me - k) % N_DEV
            pltpu.make_async_remote_copy(
                src_ref=ag_send, dst_ref=ag_recv.at[s],
                send_sem=ag_ssem.at[k - 1], recv_sem=ag_rsem.at[k - 1],
                device_id=(me,), device_id_type=pl.DeviceIdType.MESH,
            ).wait_recv()

        out_ref[...] = ag_recv[...].transpose(1, 0, 2, 3).reshape(B, SQ, DMODEL)

    out_shape = jax.ShapeDtypeStruct((B, SQ, DMODEL), jnp.float32)
    return pl.pallas_call(
        body,
        out_shape=out_shape,
        in_specs=[
            pl.BlockSpec(memory_space=pltpu.VMEM),
            pl.BlockSpec(memory_space=pltpu.VMEM),
            pl.BlockSpec(memory_space=pl.ANY),
            pl.BlockSpec(memory_space=pl.ANY),
            pl.BlockSpec(memory_space=pltpu.VMEM),
        ],
        out_specs=pl.BlockSpec(memory_space=pltpu.VMEM),
        scratch_shapes=[
            pltpu.VMEM((N_DEV, 2, B, SKV_PER, HQ_PER * DH), jnp.float32),
            pltpu.VMEM((N_DEV, 2, B, SKV_PER, HQ_PER * DH), jnp.float32),
            pltpu.VMEM((N_DEV, B, SQ_PER, DMODEL), jnp.float32),
            pltpu.VMEM((N_DEV, B, SQ_PER, DMODEL), jnp.float32),
            pltpu.VMEM((B, SQ_PER, DMODEL), jnp.float32),
            pltpu.VMEM((N_DEV, B, SQ_PER, DMODEL), jnp.float32),
            pltpu.SemaphoreType.DMA((N_DEV - 1,)),
            pltpu.SemaphoreType.DMA((N_DEV - 1,)),
            pltpu.SemaphoreType.DMA((N_DEV - 1,)),
            pltpu.SemaphoreType.DMA((N_DEV - 1,)),
            pltpu.SemaphoreType.DMA((N_DEV - 1,)),
            pltpu.SemaphoreType.DMA((N_DEV - 1,)),
            pltpu.SemaphoreType.DMA((N_DEV,)),
            pltpu.SemaphoreType.DMA,
        ],
        compiler_params=pltpu.CompilerParams(
            collective_id=0, vmem_limit_bytes=63 * 1024 * 1024
        ),
    )(
        x, Wq,
        K_ext.reshape(B, SKV_PER, 64 * DH),
        V_ext.reshape(B, SKV_PER, 64 * DH),
        Wo,
    )


# device time: 305282 ns/iter; 1.1157x vs baseline; 1.0380x over previous
import numpy as np
import jax
import jax.numpy as jnp
from jax import lax
from jax.experimental import pallas as pl
from jax.experimental.pallas import tpu as pltpu

N_DEV = 16
B, SQ, DMODEL = 2, 256, 512
HQ_PER, DH = 4, 64
SKV_PER = 256
SQ_PER = SQ // N_DEV


def _chunk_mask(s: int) -> np.ndarray:
    qb = np.arange(SQ) // 64
    kb = 4 * s + np.arange(SKV_PER) // 64
    return (
        (qb[:, None] == kb[None, :])
        | (kb[None, :] == 0)
        | ((qb[:, None] + kb[None, :]) % 3 == 0)
    )


def kernel(x, Wq, K_ext, V_ext, Wo):
    def body(x_ref, wq_ref, k_ref, v_ref, wo_ref, out_ref,
             kv_send, kv_recv, rs_send, rs_recv, ag_send, ag_recv,
             kv_ssem, kv_rsem, rs_ssem, rs_rsem, ag_ssem, ag_rsem,
             pack_sems, loc_sem):
        me = lax.axis_index("i")

        bar = pltpu.get_barrier_semaphore()
        for k in range(1, N_DEV):
            pl.semaphore_signal(
                bar, inc=1, device_id=((me + k) % N_DEV,),
                device_id_type=pl.DeviceIdType.MESH,
            )
        pl.semaphore_wait(bar, N_DEV - 1)

        packs = []
        for k in range(N_DEV):
            d = (me + k) % N_DEV
            cpk = pltpu.make_async_copy(
                k_ref.at[:, :, pl.ds(256 * d, 256)], kv_send.at[d, 0],
                pack_sems.at[k])
            cpv = pltpu.make_async_copy(
                v_ref.at[:, :, pl.ds(256 * d, 256)], kv_send.at[d, 1],
                pack_sems.at[k])
            cpk.start()
            cpv.start()
            packs.append((cpk, cpv))

        xq = x_ref[...].reshape(B * SQ, DMODEL) @ wq_ref[...]
        BH = B * HQ_PER
        Qh = xq.reshape(B, SQ, HQ_PER, DH).transpose(0, 2, 1, 3).reshape(BH, SQ, DH)

        kv_rdmas = []
        for k in range(1, N_DEV):
            d = (me + k) % N_DEV
            packs[k][0].wait()
            packs[k][1].wait()
            r = pltpu.make_async_remote_copy(
                src_ref=kv_send.at[d], dst_ref=kv_recv.at[me],
                send_sem=kv_ssem.at[k - 1], recv_sem=kv_rsem.at[me],
                device_id=(d,), device_id_type=pl.DeviceIdType.MESH,
            )
            r.start()
            kv_rdmas.append(r)

        packs[0][0].wait()
        packs[0][1].wait()
        pltpu.make_async_copy(
            kv_send.at[me], kv_recv.at[me], kv_rsem.at[me]
        ).start()

        m_run = jnp.full((BH, SQ, 1), -1e30, jnp.float32)
        l_run = jnp.zeros((BH, SQ, 1), jnp.float32)
        acc = jnp.zeros((BH, SQ, DH), jnp.float32)

        def flash_chunk(s, carry):
            m_run, l_run, acc = carry
            Kc = (kv_recv[s, 0]
                  .reshape(B, SKV_PER, HQ_PER, DH)
                  .transpose(0, 2, 1, 3).reshape(BH, SKV_PER, DH))
            Vc = (kv_recv[s, 1]
                  .reshape(B, SKV_PER, HQ_PER, DH)
                  .transpose(0, 2, 1, 3).reshape(BH, SKV_PER, DH))
            sc = lax.dot_general(
                Qh, Kc, (((2,), (2,)), ((0,), (0,)))
            ) * 0.125
            qb = lax.broadcasted_iota(jnp.int32, (SQ, SKV_PER), 0) // 64
            kb = 4 * s + lax.broadcasted_iota(jnp.int32, (SQ, SKV_PER), 1) // 64
            msk = (qb == kb) | (kb == 0) | ((qb + kb) % 3 == 0)
            sc = jnp.where(msk[None], sc, -1e9)
            m_new = jnp.maximum(m_run, sc.max(axis=-1, keepdims=True))
            alpha = jnp.exp(m_run - m_new)
            p = jnp.exp(sc - m_new)
            l_new = l_run * alpha + p.sum(axis=-1, keepdims=True)
            acc_new = acc * alpha + lax.dot_general(
                p, Vc, (((2,), (1,)), ((0,), (0,)))
            )
            return m_new, l_new, acc_new

        carry = (m_run, l_run, acc)
        for s in range(N_DEV):
            pltpu.make_async_remote_copy(
                src_ref=kv_send.at[s], dst_ref=kv_recv.at[s],
                send_sem=kv_ssem.at[0], recv_sem=kv_rsem.at[s],
                device_id=(me,), device_id_type=pl.DeviceIdType.MESH,
            ).wait_recv()
            carry = flash_chunk(s, carry)
        m_run, l_run, acc = carry
        for r in kv_rdmas:
            r.wait_send()

        ctx = (acc / l_run).reshape(B, HQ_PER, SQ, DH)
        ctx = ctx.transpose(0, 2, 1, 3).reshape(B * SQ, HQ_PER * DH)
        P = ctx @ wo_ref[...]

        rs_send[...] = P.reshape(B, N_DEV, SQ_PER, DMODEL).transpose(1, 0, 2, 3)
        cp = pltpu.make_async_copy(rs_send.at[me], rs_recv.at[me], loc_sem)
        cp.start()
        cp.wait()
        rs_rdmas = []
        for k in range(1, N_DEV):
            d = (me + k) % N_DEV
            r = pltpu.make_async_remote_copy(
                src_ref=rs_send.at[d], dst_ref=rs_recv.at[me],
                send_sem=rs_ssem.at[k - 1], recv_sem=rs_rsem.at[k - 1],
                device_id=(d,), device_id_type=pl.DeviceIdType.MESH,
            )
            r.start()
            rs_rdmas.append(r)
        for r in rs_rdmas:
            r.wait_send()
        for k in range(1, N_DEV):
            s = (me - k) % N_DEV
            pltpu.make_async_remote_copy(
                src_ref=rs_send.at[s], dst_ref=rs_recv.at[s],
                send_sem=rs_ssem.at[k - 1], recv_sem=rs_rsem.at[k - 1],
                device_id=(me,), device_id_type=pl.DeviceIdType.MESH,
            ).wait_recv()
        Rsum = rs_recv[...].sum(axis=0)

        ag_send[...] = Rsum
        cp = pltpu.make_async_copy(ag_send, ag_recv.at[me], loc_sem)
        cp.start()
        cp.wait()
        ag_rdmas = []
        for k in range(1, N_DEV):
            d = (me + k) % N_DEV
            r = pltpu.make_async_remote_copy(
                src_ref=ag_send, dst_ref=ag_recv.at[me],
                send_sem=ag_ssem.at[k - 1], recv_sem=ag_rsem.at[k - 1],
                device_id=(d,), device_id_type=pl.DeviceIdType.MESH,
            )
            r.start()
            ag_rdmas.append(r)
        for r in ag_rdmas:
            r.wait_send()
        for k in range(1, N_DEV):
            s = (me - k) % N_DEV
            pltpu.make_async_remote_copy(
                src_ref=ag_send, dst_ref=ag_recv.at[s],
                send_sem=ag_ssem.at[k - 1], recv_sem=ag_rsem.at[k - 1],
                device_id=(me,), device_id_type=pl.DeviceIdType.MESH,
            ).wait_recv()

        out_ref[...] = ag_recv[...].transpose(1, 0, 2, 3).reshape(B, SQ, DMODEL)

    out_shape = jax.ShapeDtypeStruct((B, SQ, DMODEL), jnp.float32)
    return pl.pallas_call(
        body,
        out_shape=out_shape,
        in_specs=[
            pl.BlockSpec(memory_space=pltpu.VMEM),
            pl.BlockSpec(memory_space=pltpu.VMEM),
            pl.BlockSpec(memory_space=pl.ANY),
            pl.BlockSpec(memory_space=pl.ANY),
            pl.BlockSpec(memory_space=pltpu.VMEM),
        ],
        out_specs=pl.BlockSpec(memory_space=pltpu.VMEM),
        scratch_shapes=[
            pltpu.VMEM((N_DEV, 2, B, SKV_PER, HQ_PER * DH), jnp.float32),
            pltpu.VMEM((N_DEV, 2, B, SKV_PER, HQ_PER * DH), jnp.float32),
            pltpu.VMEM((N_DEV, B, SQ_PER, DMODEL), jnp.float32),
            pltpu.VMEM((N_DEV, B, SQ_PER, DMODEL), jnp.float32),
            pltpu.VMEM((B, SQ_PER, DMODEL), jnp.float32),
            pltpu.VMEM((N_DEV, B, SQ_PER, DMODEL), jnp.float32),
            pltpu.SemaphoreType.DMA((N_DEV - 1,)),
            pltpu.SemaphoreType.DMA((N_DEV,)),
            pltpu.SemaphoreType.DMA((N_DEV - 1,)),
            pltpu.SemaphoreType.DMA((N_DEV - 1,)),
            pltpu.SemaphoreType.DMA((N_DEV - 1,)),
            pltpu.SemaphoreType.DMA((N_DEV - 1,)),
            pltpu.SemaphoreType.DMA((N_DEV,)),
            pltpu.SemaphoreType.DMA,
        ],
        compiler_params=pltpu.CompilerParams(
            collective_id=0, vmem_limit_bytes=63 * 1024 * 1024
        ),
    )(
        x, Wq,
        K_ext.reshape(B, SKV_PER, 64 * DH),
        V_ext.reshape(B, SKV_PER, 64 * DH),
        Wo,
    )


# device time: 301942 ns/iter; 1.1280x vs baseline; 1.0111x over previous
import numpy as np
import jax
import jax.numpy as jnp
from jax import lax
from jax.experimental import pallas as pl
from jax.experimental.pallas import tpu as pltpu

N_DEV = 16
B, SQ, DMODEL = 2, 256, 512
HQ_PER, DH = 4, 64
SKV_PER = 256
SQ_PER = SQ // N_DEV


def _chunk_mask(s: int) -> np.ndarray:
    qb = np.arange(SQ) // 64
    kb = 4 * s + np.arange(SKV_PER) // 64
    return (
        (qb[:, None] == kb[None, :])
        | (kb[None, :] == 0)
        | ((qb[:, None] + kb[None, :]) % 3 == 0)
    )


def kernel(x, Wq, K_ext, V_ext, Wo):
    def body(x_ref, wq_ref, k_ref, v_ref, wo_ref, out_ref,
             kv_send, kv_recv, rs_send, rs_recv, ag_send, ag_recv,
             kv_ssem, kv_rsem, rs_ssem, rs_rsem, ag_ssem, ag_rsem,
             pack_sems, loc_sem):
        me = lax.axis_index("i")

        bar = pltpu.get_barrier_semaphore()
        for k in range(1, N_DEV):
            pl.semaphore_signal(
                bar, inc=1, device_id=((me + k) % N_DEV,),
                device_id_type=pl.DeviceIdType.MESH,
            )
        pl.semaphore_wait(bar, N_DEV - 1)

        packs = []
        for k in range(N_DEV):
            d = (me + k) % N_DEV
            cpk = pltpu.make_async_copy(
                k_ref.at[:, :, pl.ds(256 * d, 256)], kv_send.at[d, 0],
                pack_sems.at[k])
            cpv = pltpu.make_async_copy(
                v_ref.at[:, :, pl.ds(256 * d, 256)], kv_send.at[d, 1],
                pack_sems.at[k])
            cpk.start()
            cpv.start()
            packs.append((cpk, cpv))

        xq = (x_ref[...].reshape(B * SQ, DMODEL) @ wq_ref[...]) * 0.125
        Qs = [xq[:, 64 * h:64 * h + 64].reshape(B, SQ, DH) for h in range(HQ_PER)]

        kv_rdmas = []
        for k in range(1, N_DEV):
            d = (me + k) % N_DEV
            packs[k][0].wait()
            packs[k][1].wait()
            r = pltpu.make_async_remote_copy(
                src_ref=kv_send.at[d], dst_ref=kv_recv.at[me],
                send_sem=kv_ssem.at[k - 1], recv_sem=kv_rsem.at[me],
                device_id=(d,), device_id_type=pl.DeviceIdType.MESH,
            )
            r.start()
            kv_rdmas.append(r)

        packs[0][0].wait()
        packs[0][1].wait()
        pltpu.make_async_copy(
            kv_send.at[me], kv_recv.at[me], kv_rsem.at[me]
        ).start()

        m_run = [jnp.full((B, SQ, 1), -1e30, jnp.float32)] * HQ_PER
        l_run = [jnp.zeros((B, SQ, 1), jnp.float32)] * HQ_PER
        acc = [jnp.zeros((B, SQ, DH), jnp.float32)] * HQ_PER

        def flash_chunk(s, m_run, l_run, acc):
            Kc = kv_recv[s, 0]
            Vc = kv_recv[s, 1]
            qb = lax.broadcasted_iota(jnp.int32, (SQ, SKV_PER), 0) // 64
            kb = 4 * s + lax.broadcasted_iota(jnp.int32, (SQ, SKV_PER), 1) // 64
            msk = (qb == kb) | (kb == 0) | ((qb + kb) % 3 == 0)
            m_n, l_n, acc_n = [], [], []
            for h in range(HQ_PER):
                Kh = Kc[:, :, 64 * h:64 * h + 64]
                Vh = Vc[:, :, 64 * h:64 * h + 64]
                sc = lax.dot_general(
                    Qs[h], Kh, (((2,), (2,)), ((0,), (0,)))
                )
                sc = jnp.where(msk[None], sc, -1e9)
                m_new = jnp.maximum(m_run[h], sc.max(axis=-1, keepdims=True))
                alpha = jnp.exp(m_run[h] - m_new)
                p = jnp.exp(sc - m_new)
                m_n.append(m_new)
                l_n.append(l_run[h] * alpha + p.sum(axis=-1, keepdims=True))
                acc_n.append(acc[h] * alpha + lax.dot_general(
                    p, Vh, (((2,), (1,)), ((0,), (0,)))
                ))
            return m_n, l_n, acc_n

        for s in range(N_DEV):
            pltpu.make_async_remote_copy(
                src_ref=kv_send.at[s], dst_ref=kv_recv.at[s],
                send_sem=kv_ssem.at[0], recv_sem=kv_rsem.at[s],
                device_id=(me,), device_id_type=pl.DeviceIdType.MESH,
            ).wait_recv()
            m_run, l_run, acc = flash_chunk(s, m_run, l_run, acc)
        for r in kv_rdmas:
            r.wait_send()

        ctx = jnp.concatenate(
            [(acc[h] / l_run[h]).reshape(B * SQ, DH) for h in range(HQ_PER)],
            axis=1,
        )
        P = ctx @ wo_ref[...]

        rs_send[...] = P.reshape(B, N_DEV, SQ_PER, DMODEL).transpose(1, 0, 2, 3)
        cp = pltpu.make_async_copy(rs_send.at[me], rs_recv.at[me], loc_sem)
        cp.start()
        cp.wait()
        rs_rdmas = []
        for k in range(1, N_DEV):
            d = (me + k) % N_DEV
            r = pltpu.make_async_remote_copy(
                src_ref=rs_send.at[d], dst_ref=rs_recv.at[me],
                send_sem=rs_ssem.at[k - 1], recv_sem=rs_rsem.at[k - 1],
                device_id=(d,), device_id_type=pl.DeviceIdType.MESH,
            )
            r.start()
            rs_rdmas.append(r)
        for r in rs_rdmas:
            r.wait_send()
        for k in range(1, N_DEV):
            s = (me - k) % N_DEV
            pltpu.make_async_remote_copy(
                src_ref=rs_send.at[s], dst_ref=rs_recv.at[s],
                send_sem=rs_ssem.at[k - 1], recv_sem=rs_rsem.at[k - 1],
                device_id=(me,), device_id_type=pl.DeviceIdType.MESH,
            ).wait_recv()
        Rsum = rs_recv[...].sum(axis=0)

        ag_send[...] = Rsum
        cp = pltpu.make_async_copy(ag_send, ag_recv.at[me], loc_sem)
        cp.start()
        cp.wait()
        ag_rdmas = []
        for k in range(1, N_DEV):
            d = (me + k) % N_DEV
            r = pltpu.make_async_remote_copy(
                src_ref=ag_send, dst_ref=ag_recv.at[me],
                send_sem=ag_ssem.at[k - 1], recv_sem=ag_rsem.at[k - 1],
                device_id=(d,), device_id_type=pl.DeviceIdType.MESH,
            )
            r.start()
            ag_rdmas.append(r)
        for r in ag_rdmas:
            r.wait_send()
        for k in range(1, N_DEV):
            s = (me - k) % N_DEV
            pltpu.make_async_remote_copy(
                src_ref=ag_send, dst_ref=ag_recv.at[s],
                send_sem=ag_ssem.at[k - 1], recv_sem=ag_rsem.at[k - 1],
                device_id=(me,), device_id_type=pl.DeviceIdType.MESH,
            ).wait_recv()

        out_ref[...] = ag_recv[...].transpose(1, 0, 2, 3).reshape(B, SQ, DMODEL)

    out_shape = jax.ShapeDtypeStruct((B, SQ, DMODEL), jnp.float32)
    return pl.pallas_call(
        body,
        out_shape=out_shape,
        in_specs=[
            pl.BlockSpec(memory_space=pltpu.VMEM),
            pl.BlockSpec(memory_space=pltpu.VMEM),
            pl.BlockSpec(memory_space=pl.ANY),
            pl.BlockSpec(memory_space=pl.ANY),
            pl.BlockSpec(memory_space=pltpu.VMEM),
        ],
        out_specs=pl.BlockSpec(memory_space=pltpu.VMEM),
        scratch_shapes=[
            pltpu.VMEM((N_DEV, 2, B, SKV_PER, HQ_PER * DH), jnp.float32),
            pltpu.VMEM((N_DEV, 2, B, SKV_PER, HQ_PER * DH), jnp.float32),
            pltpu.VMEM((N_DEV, B, SQ_PER, DMODEL), jnp.float32),
            pltpu.VMEM((N_DEV, B, SQ_PER, DMODEL), jnp.float32),
            pltpu.VMEM((B, SQ_PER, DMODEL), jnp.float32),
            pltpu.VMEM((N_DEV, B, SQ_PER, DMODEL), jnp.float32),
            pltpu.SemaphoreType.DMA((N_DEV - 1,)),
            pltpu.SemaphoreType.DMA((N_DEV,)),
            pltpu.SemaphoreType.DMA((N_DEV - 1,)),
            pltpu.SemaphoreType.DMA((N_DEV - 1,)),
            pltpu.SemaphoreType.DMA((N_DEV - 1,)),
            pltpu.SemaphoreType.DMA((N_DEV - 1,)),
            pltpu.SemaphoreType.DMA((N_DEV,)),
            pltpu.SemaphoreType.DMA,
        ],
        compiler_params=pltpu.CompilerParams(
            collective_id=0, vmem_limit_bytes=63 * 1024 * 1024
        ),
    )(
        x, Wq,
        K_ext.reshape(B, SKV_PER, 64 * DH),
        V_ext.reshape(B, SKV_PER, 64 * DH),
        Wo,
    )


# device time: 184377 ns/iter; 1.8473x vs baseline; 1.6376x over previous
import numpy as np
import jax
import jax.numpy as jnp
from jax import lax
from jax.experimental import pallas as pl
from jax.experimental.pallas import tpu as pltpu

N_DEV = 16
B, SQ, DMODEL = 2, 256, 512
HQ_PER, DH = 4, 64
SKV_PER = 256
SQ_PER = SQ // N_DEV


def _chunk_mask(s: int) -> np.ndarray:
    qb = np.arange(SQ) // 64
    kb = 4 * s + np.arange(SKV_PER) // 64
    return (
        (qb[:, None] == kb[None, :])
        | (kb[None, :] == 0)
        | ((qb[:, None] + kb[None, :]) % 3 == 0)
    )


def kernel(x, Wq, K_ext, V_ext, Wo):
    def body(x_ref, wq_ref, k_ref, v_ref, wo_ref, out_ref,
             kv_send, kv_recv, rs_send, rs_recv, ag_send, ag_recv,
             kv_ssem, kv_rsem, rs_ssem, rs_rsem, ag_ssem, ag_rsem,
             pack_sems, loc_sem):
        me = lax.axis_index("i")

        bar = pltpu.get_barrier_semaphore()
        for k in range(1, N_DEV):
            pl.semaphore_signal(
                bar, inc=1, device_id=((me + k) % N_DEV,),
                device_id_type=pl.DeviceIdType.MESH,
            )
        pl.semaphore_wait(bar, N_DEV - 1)

        packs = []
        for k in range(N_DEV):
            d = (me + k) % N_DEV
            cpk = pltpu.make_async_copy(
                k_ref.at[:, :, pl.ds(256 * d, 256)], kv_send.at[d, 0],
                pack_sems.at[k])
            cpv = pltpu.make_async_copy(
                v_ref.at[:, :, pl.ds(256 * d, 256)], kv_send.at[d, 1],
                pack_sems.at[k])
            cpk.start()
            cpv.start()
            packs.append((cpk, cpv))

        xq = (x_ref[...].reshape(B * SQ, DMODEL) @ wq_ref[...]) * 0.125
        Qs = [
            xq[:, 64 * h:64 * h + 64].reshape(B, SQ, DH).astype(jnp.bfloat16)
            for h in range(HQ_PER)
        ]

        kv_rdmas = []
        for k in range(1, N_DEV):
            d = (me + k) % N_DEV
            packs[k][0].wait()
            packs[k][1].wait()
            r = pltpu.make_async_remote_copy(
                src_ref=kv_send.at[d], dst_ref=kv_recv.at[me],
                send_sem=kv_ssem.at[k - 1], recv_sem=kv_rsem.at[me],
                device_id=(d,), device_id_type=pl.DeviceIdType.MESH,
            )
            r.start()
            kv_rdmas.append(r)

        packs[0][0].wait()
        packs[0][1].wait()
        pltpu.make_async_copy(
            kv_send.at[me], kv_recv.at[me], kv_rsem.at[me]
        ).start()

        m_run = [jnp.full((B, SQ, 1), -1e30, jnp.float32)] * HQ_PER
        l_run = [jnp.zeros((B, SQ, 1), jnp.float32)] * HQ_PER
        acc = [jnp.zeros((B, SQ, DH), jnp.float32)] * HQ_PER

        def flash_chunk(s, m_run, l_run, acc):
            Kc = kv_recv[s, 0]
            Vc = kv_recv[s, 1]
            qb = lax.broadcasted_iota(jnp.int32, (SQ, SKV_PER), 0) // 64
            kb = 4 * s + lax.broadcasted_iota(jnp.int32, (SQ, SKV_PER), 1) // 64
            msk = (qb == kb) | (kb == 0) | ((qb + kb) % 3 == 0)
            m_n, l_n, acc_n = [], [], []
            for h in range(HQ_PER):
                Kh = Kc[:, :, 64 * h:64 * h + 64]
                Vh = Vc[:, :, 64 * h:64 * h + 64]
                sc = lax.dot_general(
                    Qs[h], Kh, (((2,), (2,)), ((0,), (0,))),
                    preferred_element_type=jnp.float32,
                )
                sc = jnp.where(msk[None], sc, -1e9)
                m_new = jnp.maximum(m_run[h], sc.max(axis=-1, keepdims=True))
                alpha = jnp.exp(m_run[h] - m_new)
                p = jnp.exp(sc - m_new)
                m_n.append(m_new)
                l_n.append(l_run[h] * alpha + p.sum(axis=-1, keepdims=True))
                acc_n.append(acc[h] * alpha + lax.dot_general(
                    p.astype(jnp.bfloat16), Vh, (((2,), (1,)), ((0,), (0,))),
                    preferred_element_type=jnp.float32,
                ))
            return m_n, l_n, acc_n

        for s in range(N_DEV):
            pltpu.make_async_remote_copy(
                src_ref=kv_send.at[s], dst_ref=kv_recv.at[s],
                send_sem=kv_ssem.at[0], recv_sem=kv_rsem.at[s],
                device_id=(me,), device_id_type=pl.DeviceIdType.MESH,
            ).wait_recv()
            m_run, l_run, acc = flash_chunk(s, m_run, l_run, acc)
        for r in kv_rdmas:
            r.wait_send()

        ctx = jnp.concatenate(
            [(acc[h] / l_run[h]).reshape(B * SQ, DH) for h in range(HQ_PER)],
            axis=1,
        )
        P = ctx @ wo_ref[...]

        rs_send[...] = P.reshape(B, N_DEV, SQ_PER, DMODEL).transpose(1, 0, 2, 3)
        cp = pltpu.make_async_copy(rs_send.at[me], rs_recv.at[me], loc_sem)
        cp.start()
        cp.wait()
        rs_rdmas = []
        for k in range(1, N_DEV):
            d = (me + k) % N_DEV
            r = pltpu.make_async_remote_copy(
                src_ref=rs_send.at[d], dst_ref=rs_recv.at[me],
                send_sem=rs_ssem.at[k - 1], recv_sem=rs_rsem.at[k - 1],
                device_id=(d,), device_id_type=pl.DeviceIdType.MESH,
            )
            r.start()
            rs_rdmas.append(r)
        for r in rs_rdmas:
            r.wait_send()
        for k in range(1, N_DEV):
            s = (me - k) % N_DEV
            pltpu.make_async_remote_copy(
                src_ref=rs_send.at[s], dst_ref=rs_recv.at[s],
                send_sem=rs_ssem.at[k - 1], recv_sem=rs_rsem.at[k - 1],
                device_id=(me,), device_id_type=pl.DeviceIdType.MESH,
            ).wait_recv()
        Rsum = rs_recv[...].sum(axis=0)

        ag_send[...] = Rsum
        cp = pltpu.make_async_copy(ag_send, ag_recv.at[me], loc_sem)
        cp.start()
        cp.wait()
        ag_rdmas = []
        for k in range(1, N_DEV):
            d = (me + k) % N_DEV
            r = pltpu.make_async_remote_copy(
                src_ref=ag_send, dst_ref=ag_recv.at[me],
                send_sem=ag_ssem.at[k - 1], recv_sem=ag_rsem.at[k - 1],
                device_id=(d,), device_id_type=pl.DeviceIdType.MESH,
            )
            r.start()
            ag_rdmas.append(r)
        for r in ag_rdmas:
            r.wait_send()
        for k in range(1, N_DEV):
            s = (me - k) % N_DEV
            pltpu.make_async_remote_copy(
                src_ref=ag_send, dst_ref=ag_recv.at[s],
                send_sem=ag_ssem.at[k - 1], recv_sem=ag_rsem.at[k - 1],
                device_id=(me,), device_id_type=pl.DeviceIdType.MESH,
            ).wait_recv()

        out_ref[...] = ag_recv[...].transpose(1, 0, 2, 3).reshape(B, SQ, DMODEL)

    out_shape = jax.ShapeDtypeStruct((B, SQ, DMODEL), jnp.float32)
    return pl.pallas_call(
        body,
        out_shape=out_shape,
        in_specs=[
            pl.BlockSpec(memory_space=pltpu.VMEM),
            pl.BlockSpec(memory_space=pltpu.VMEM),
            pl.BlockSpec(memory_space=pltpu.VMEM),
            pl.BlockSpec(memory_space=pltpu.VMEM),
            pl.BlockSpec(memory_space=pltpu.VMEM),
        ],
        out_specs=pl.BlockSpec(memory_space=pltpu.VMEM),
        scratch_shapes=[
            pltpu.VMEM((N_DEV, 2, B, SKV_PER, HQ_PER * DH), jnp.bfloat16),
            pltpu.VMEM((N_DEV, 2, B, SKV_PER, HQ_PER * DH), jnp.bfloat16),
            pltpu.VMEM((N_DEV, B, SQ_PER, DMODEL), jnp.float32),
            pltpu.VMEM((N_DEV, B, SQ_PER, DMODEL), jnp.float32),
            pltpu.VMEM((B, SQ_PER, DMODEL), jnp.float32),
            pltpu.VMEM((N_DEV, B, SQ_PER, DMODEL), jnp.float32),
            pltpu.SemaphoreType.DMA((N_DEV - 1,)),
            pltpu.SemaphoreType.DMA((N_DEV,)),
            pltpu.SemaphoreType.DMA((N_DEV - 1,)),
            pltpu.SemaphoreType.DMA((N_DEV - 1,)),
            pltpu.SemaphoreType.DMA((N_DEV - 1,)),
            pltpu.SemaphoreType.DMA((N_DEV - 1,)),
            pltpu.SemaphoreType.DMA((N_DEV,)),
            pltpu.SemaphoreType.DMA,
        ],
        compiler_params=pltpu.CompilerParams(
            collective_id=0, vmem_limit_bytes=63 * 1024 * 1024
        ),
    )(
        x, Wq,
        K_ext.reshape(B, SKV_PER, 64 * DH).astype(jnp.bfloat16),
        V_ext.reshape(B, SKV_PER, 64 * DH).astype(jnp.bfloat16),
        Wo,
    )


# device time: 175162 ns/iter; 1.9444x vs baseline; 1.0526x over previous
import numpy as np
import jax
import jax.numpy as jnp
from jax import lax
from jax.experimental import pallas as pl
from jax.experimental.pallas import tpu as pltpu

N_DEV = 16
B, SQ, DMODEL = 2, 256, 512
HQ_PER, DH = 4, 64
SKV_PER = 256
SQ_PER = SQ // N_DEV


def _chunk_mask(s: int) -> np.ndarray:
    qb = np.arange(SQ) // 64
    kb = 4 * s + np.arange(SKV_PER) // 64
    return (
        (qb[:, None] == kb[None, :])
        | (kb[None, :] == 0)
        | ((qb[:, None] + kb[None, :]) % 3 == 0)
    )


def kernel(x, Wq, K_ext, V_ext, Wo):
    def body(x_ref, wq_ref, k_ref, v_ref, wo_ref, out_ref,
             kv_send, kv_recv, rs_send, rs_recv, ag_send, ag_recv,
             kv_ssem, kv_rsem, rs_ssem, rs_rsem, ag_ssem, ag_rsem,
             pack_sems, loc_sem):
        me = lax.axis_index("i")

        bar = pltpu.get_barrier_semaphore()
        for k in range(1, N_DEV):
            pl.semaphore_signal(
                bar, inc=1, device_id=((me + k) % N_DEV,),
                device_id_type=pl.DeviceIdType.MESH,
            )
        pl.semaphore_wait(bar, N_DEV - 1)

        packs = []
        for k in range(N_DEV):
            d = (me + k) % N_DEV
            cpk = pltpu.make_async_copy(
                k_ref.at[:, :, pl.ds(256 * d, 256)], kv_send.at[d, 0],
                pack_sems.at[k])
            cpv = pltpu.make_async_copy(
                v_ref.at[:, :, pl.ds(256 * d, 256)], kv_send.at[d, 1],
                pack_sems.at[k])
            cpk.start()
            cpv.start()
            packs.append((cpk, cpv))

        xq = (x_ref[...].reshape(B * SQ, DMODEL) @ wq_ref[...]) * 0.125
        Qs = [
            xq[:, 64 * h:64 * h + 64].reshape(B, SQ, DH).astype(jnp.bfloat16)
            for h in range(HQ_PER)
        ]

        kv_rdmas = []
        for k in range(1, N_DEV):
            d = (me + k) % N_DEV
            packs[k][0].wait()
            packs[k][1].wait()
            r = pltpu.make_async_remote_copy(
                src_ref=kv_send.at[d], dst_ref=kv_recv.at[me],
                send_sem=kv_ssem.at[k - 1], recv_sem=kv_rsem.at[me],
                device_id=(d,), device_id_type=pl.DeviceIdType.MESH,
            )
            r.start()
            kv_rdmas.append(r)

        packs[0][0].wait()
        packs[0][1].wait()
        pltpu.make_async_copy(
            kv_send.at[me], kv_recv.at[me], kv_rsem.at[me]
        ).start()

        m_run = [jnp.full((B, SQ, 1), -1e30, jnp.float32)] * HQ_PER
        l_run = [jnp.zeros((B, SQ, 1), jnp.float32)] * HQ_PER
        acc = [jnp.zeros((B, SQ, DH), jnp.float32)] * HQ_PER

        def flash_chunk(s, m_run, l_run, acc):
            Kc = kv_recv[s, 0]
            Vc = kv_recv[s, 1]
            qb = lax.broadcasted_iota(jnp.int32, (SQ, SKV_PER), 0) // 64
            kb = 4 * s + lax.broadcasted_iota(jnp.int32, (SQ, SKV_PER), 1) // 64
            msk = (qb == kb) | (kb == 0) | ((qb + kb) % 3 == 0)
            m_n, l_n, acc_n = [], [], []
            for h in range(HQ_PER):
                Kh = Kc[:, :, 64 * h:64 * h + 64]
                Vh = Vc[:, :, 64 * h:64 * h + 64]
                sc = lax.dot_general(
                    Qs[h], Kh, (((2,), (2,)), ((0,), (0,))),
                    preferred_element_type=jnp.float32,
                )
                sc = jnp.where(msk[None], sc, -1e9)
                m_new = jnp.maximum(m_run[h], sc.max(axis=-1, keepdims=True))
                alpha = jnp.exp(m_run[h] - m_new)
                p = jnp.exp(sc - m_new)
                m_n.append(m_new)
                l_n.append(l_run[h] * alpha + p.sum(axis=-1, keepdims=True))
                acc_n.append(acc[h] * alpha + lax.dot_general(
                    p.astype(jnp.bfloat16), Vh, (((2,), (1,)), ((0,), (0,))),
                    preferred_element_type=jnp.float32,
                ))
            return m_n, l_n, acc_n

        for s in range(N_DEV):
            pltpu.make_async_remote_copy(
                src_ref=kv_send.at[s], dst_ref=kv_recv.at[s],
                send_sem=kv_ssem.at[0], recv_sem=kv_rsem.at[s],
                device_id=(me,), device_id_type=pl.DeviceIdType.MESH,
            ).wait_recv()
            m_run, l_run, acc = flash_chunk(s, m_run, l_run, acc)
        for r in kv_rdmas:
            r.wait_send()

        ctx = jnp.concatenate(
            [(acc[h] / l_run[h]).reshape(B * SQ, DH) for h in range(HQ_PER)],
            axis=1,
        )
        P = ctx @ wo_ref[...]

        rs_send[...] = (P.reshape(B, N_DEV, SQ_PER, DMODEL)
                        .transpose(1, 0, 2, 3).astype(jnp.bfloat16))
        cp = pltpu.make_async_copy(rs_send.at[me], rs_recv.at[me], loc_sem)
        cp.start()
        cp.wait()
        rs_rdmas = []
        for k in range(1, N_DEV):
            d = (me + k) % N_DEV
            r = pltpu.make_async_remote_copy(
                src_ref=rs_send.at[d], dst_ref=rs_recv.at[me],
                send_sem=rs_ssem.at[k - 1], recv_sem=rs_rsem.at[k - 1],
                device_id=(d,), device_id_type=pl.DeviceIdType.MESH,
            )
            r.start()
            rs_rdmas.append(r)
        for r in rs_rdmas:
            r.wait_send()
        for k in range(1, N_DEV):
            s = (me - k) % N_DEV
            pltpu.make_async_remote_copy(
                src_ref=rs_send.at[s], dst_ref=rs_recv.at[s],
                send_sem=rs_ssem.at[k - 1], recv_sem=rs_rsem.at[k - 1],
                device_id=(me,), device_id_type=pl.DeviceIdType.MESH,
            ).wait_recv()
        Rsum = rs_recv[...].astype(jnp.float32).sum(axis=0)

        ag_send[...] = Rsum.astype(jnp.bfloat16)
        cp = pltpu.make_async_copy(ag_send, ag_recv.at[me], loc_sem)
        cp.start()
        cp.wait()
        ag_rdmas = []
        for k in range(1, N_DEV):
            d = (me + k) % N_DEV
            r = pltpu.make_async_remote_copy(
                src_ref=ag_send, dst_ref=ag_recv.at[me],
                send_sem=ag_ssem.at[k - 1], recv_sem=ag_rsem.at[k - 1],
                device_id=(d,), device_id_type=pl.DeviceIdType.MESH,
            )
            r.start()
            ag_rdmas.append(r)
        for r in ag_rdmas:
            r.wait_send()
        for k in range(1, N_DEV):
            s = (me - k) % N_DEV
            pltpu.make_async_remote_copy(
                src_ref=ag_send, dst_ref=ag_recv.at[s],
                send_sem=ag_ssem.at[k - 1], recv_sem=ag_rsem.at[k - 1],
                device_id=(me,), device_id_type=pl.DeviceIdType.MESH,
            ).wait_recv()

        out_ref[...] = (ag_recv[...].astype(jnp.float32)
                        .transpose(1, 0, 2, 3).reshape(B, SQ, DMODEL))

    out_shape = jax.ShapeDtypeStruct((B, SQ, DMODEL), jnp.float32)
    return pl.pallas_call(
        body,
        out_shape=out_shape,
        in_specs=[
            pl.BlockSpec(memory_space=pltpu.VMEM),
            pl.BlockSpec(memory_space=pltpu.VMEM),
            pl.BlockSpec(memory_space=pltpu.VMEM),
            pl.BlockSpec(memory_space=pltpu.VMEM),
            pl.BlockSpec(memory_space=pltpu.VMEM),
        ],
        out_specs=pl.BlockSpec(memory_space=pltpu.VMEM),
        scratch_shapes=[
            pltpu.VMEM((N_DEV, 2, B, SKV_PER, HQ_PER * DH), jnp.bfloat16),
            pltpu.VMEM((N_DEV, 2, B, SKV_PER, HQ_PER * DH), jnp.bfloat16),
            pltpu.VMEM((N_DEV, B, SQ_PER, DMODEL), jnp.bfloat16),
            pltpu.VMEM((N_DEV, B, SQ_PER, DMODEL), jnp.bfloat16),
            pltpu.VMEM((B, SQ_PER, DMODEL), jnp.bfloat16),
            pltpu.VMEM((N_DEV, B, SQ_PER, DMODEL), jnp.bfloat16),
            pltpu.SemaphoreType.DMA((N_DEV - 1,)),
            pltpu.SemaphoreType.DMA((N_DEV,)),
            pltpu.SemaphoreType.DMA((N_DEV - 1,)),
            pltpu.SemaphoreType.DMA((N_DEV - 1,)),
            pltpu.SemaphoreType.DMA((N_DEV - 1,)),
            pltpu.SemaphoreType.DMA((N_DEV - 1,)),
            pltpu.SemaphoreType.DMA((N_DEV,)),
            pltpu.SemaphoreType.DMA,
        ],
        compiler_params=pltpu.CompilerParams(
            collective_id=0, vmem_limit_bytes=63 * 1024 * 1024
        ),
    )(
        x, Wq,
        K_ext.reshape(B, SKV_PER, 64 * DH).astype(jnp.bfloat16),
        V_ext.reshape(B, SKV_PER, 64 * DH).astype(jnp.bfloat16),
        Wo,
    )
